# baseline (speedup 1.0000x reference)
import functools

import jax
import jax.numpy as jnp
from jax import lax
from jax.experimental import pallas as pl
from jax.experimental.pallas import tpu as pltpu
from jax.experimental.pallas import tpu_sc as plsc

_NW = 32
_L = 16


def _emb_body(bw, ids_t, table4, out_t, idst, pid_a, col_a,
              rows0, rows1, blk0, blk1, g0, g1, o0, o1):
    s, btot = ids_t.shape
    d = blk0.shape[0]
    wid = lax.axis_index("s") * 2 + lax.axis_index("c")
    b0 = wid * bw
    n_tiles = s // 8

    def decode_tile(t, carry):
        pltpu.sync_copy(ids_t.at[pl.ds(t * 8, 8), pl.ds(b0, bw)], idst)
        for r in range(8):
            for l in range(bw // _L):
                v = idst[r, pl.ds(l * _L, _L)]
                pid_a[t * 8 + r, pl.ds(l * _L, _L)] = lax.shift_right_logical(v, 2)
                col_a[t * 8 + r, pl.ds(l * _L, _L)] = (v & 3) * d
        return carry

    lax.fori_loop(0, n_tiles, decode_tile, 0, unroll=False)

    def g_copy(r, rows, sem):
        return pltpu.make_async_copy(table4.at[pid_a.at[r]], rows, sem)

    def s_copy(r, blk, sem):
        return pltpu.make_async_copy(blk, out_t.at[r, :, pl.ds(b0, bw)], sem)

    def extract(r, rows, blk):
        cols = [col_a[r, pl.ds(l * _L, _L)] for l in range(bw // _L)]
        js = [lax.iota(jnp.int32, _L) + l * _L for l in range(bw // _L)]

        def fbody(f, carry):
            for l in range(bw // _L):
                blk[f, pl.ds(l * _L, _L)] = plsc.load_gather(
                    rows, [js[l], carry[l]])
            return tuple(c + 1 for c in carry)

        lax.fori_loop(0, d, fbody, tuple(cols), unroll=False)

    g_copy(0, rows0, g0).start()
    g_copy(1, rows1, g1).start()

    def body(i, carry):
        r = 2 * i
        g_copy(r, rows0, g0).wait()

        @pl.when(i > 0)
        def _():
            s_copy(r - 2, blk0, o0).wait()

        extract(r, rows0, blk0)
        s_copy(r, blk0, o0).start()

        @pl.when(i < (s // 2) - 1)
        def _():
            g_copy(r + 2, rows0, g0).start()

        g_copy(r + 1, rows1, g1).wait()

        @pl.when(i > 0)
        def _():
            s_copy(r - 1, blk1, o1).wait()

        extract(r + 1, rows1, blk1)
        s_copy(r + 1, blk1, o1).start()

        @pl.when(i < (s // 2) - 1)
        def _():
            g_copy(r + 3, rows1, g1).start()

        return carry

    lax.fori_loop(0, s // 2, body, 0, unroll=False)
    s_copy(0, blk0, o0).wait()
    s_copy(1, blk1, o1).wait()


def kernel(token_ids, table):
    b, s = token_ids.shape
    v, d = table.shape
    assert b % (_NW * _L) == 0 and s % 8 == 0 and v % 4 == 0
    bw = b // _NW
    pack = 128 // d
    assert pack * d == 128

    ids_t = token_ids.T
    table4 = jnp.reshape(table, (v // pack, 128))

    mesh = plsc.VectorSubcoreMesh(core_axis_name="c", subcore_axis_name="s")
    k = pl.kernel(
        functools.partial(_emb_body, bw),
        out_type=jax.ShapeDtypeStruct((s, d, b), jnp.float32),
        mesh=mesh,
        scratch_types=[
            pltpu.VMEM((8, bw), jnp.int32),
            pltpu.VMEM((s, bw), jnp.int32),
            pltpu.VMEM((s, bw), jnp.int32),
            pltpu.VMEM((bw, 128), jnp.float32),
            pltpu.VMEM((bw, 128), jnp.float32),
            pltpu.VMEM((d, bw), jnp.float32),
            pltpu.VMEM((d, bw), jnp.float32),
            pltpu.SemaphoreType.DMA,
            pltpu.SemaphoreType.DMA,
            pltpu.SemaphoreType.DMA,
            pltpu.SemaphoreType.DMA,
        ],
        compiler_params=pltpu.CompilerParams(
            use_tc_tiling_on_sc=True, needs_layout_passes=False),
    )
    out_t = k(ids_t, table4)
    return jnp.transpose(out_t, (2, 0, 1))


# speedup vs baseline: 1.1534x; 1.1534x over previous
import functools

import jax
import jax.numpy as jnp
from jax import lax
from jax.experimental import pallas as pl
from jax.experimental.pallas import tpu as pltpu
from jax.experimental.pallas import tpu_sc as plsc

_NW = 32
_CH = 1280


def _gather_body(per_w, n_pairs, ids_hbm, table_hbm, out_hbm,
                 idx_v, rows0, rows1, gs0, gs1, os0, os1):
    ch = _CH
    wid = lax.axis_index("s") * 2 + lax.axis_index("c")
    base = wid * per_w
    pltpu.sync_copy(ids_hbm.at[pl.ds(base, per_w)], idx_v)

    def g_copy(g, rows, sem):
        return pltpu.make_async_copy(
            table_hbm.at[idx_v.at[pl.ds(g * ch, ch)]], rows, sem)

    def s_copy(g, rows, sem):
        return pltpu.make_async_copy(
            rows, out_hbm.at[pl.ds(base + g * ch, ch)], sem)

    g_copy(0, rows0, gs0).start()

    def body(i, carry):
        a = 2 * i
        g_copy(a, rows0, gs0).wait()
        s_copy(a, rows0, os0).start()

        @pl.when(i > 0)
        def _():
            s_copy(a - 1, rows1, os1).wait()

        g_copy(a + 1, rows1, gs1).start()
        g_copy(a + 1, rows1, gs1).wait()
        s_copy(a + 1, rows1, os1).start()

        @pl.when(i + 1 < n_pairs)
        def _():
            s_copy(a, rows0, os0).wait()
            g_copy(a + 2, rows0, gs0).start()

        return carry

    lax.fori_loop(0, n_pairs, body, 0, unroll=False)
    s_copy(0, rows0, os0).wait()
    s_copy(0, rows1, os1).wait()


def kernel(token_ids, table):
    b, s = token_ids.shape
    _, d = table.shape
    n = b * s
    assert n % (_NW * 2 * _CH) == 0
    per_w = n // _NW
    n_pairs = per_w // (2 * _CH)

    flat_ids = token_ids.reshape(n).astype(jnp.int32)
    mesh = plsc.VectorSubcoreMesh(core_axis_name="c", subcore_axis_name="s")
    k = pl.kernel(
        functools.partial(_gather_body, per_w, n_pairs),
        out_type=jax.ShapeDtypeStruct((n, d), jnp.float32),
        mesh=mesh,
        scratch_types=[
            pltpu.VMEM((per_w,), jnp.int32),
            pltpu.VMEM((_CH, d), jnp.float32),
            pltpu.VMEM((_CH, d), jnp.float32),
            pltpu.SemaphoreType.DMA,
            pltpu.SemaphoreType.DMA,
            pltpu.SemaphoreType.DMA,
            pltpu.SemaphoreType.DMA,
        ],
        compiler_params=pltpu.CompilerParams(use_tc_tiling_on_sc=False),
    )
    out = k(flat_ids, table)
    return out.reshape(b, s, d)


# speedup vs baseline: 1.1570x; 1.0031x over previous
import functools

import jax
import jax.numpy as jnp
from jax import lax
from jax.experimental import pallas as pl
from jax.experimental.pallas import tpu as pltpu
from jax.experimental.pallas import tpu_sc as plsc

_NW = 32
_CH = 800
_NB = 4


def _gather_body(per_w, ids_hbm, table_hbm, out_hbm,
                 idx_v, rows, gsems, osems):
    ch = _CH
    n_chunks = per_w // ch
    wid = lax.axis_index("s") * 2 + lax.axis_index("c")
    base = wid * per_w
    pltpu.sync_copy(ids_hbm.at[pl.ds(base, per_w)], idx_v)

    def g_copy(g, k):
        return pltpu.make_async_copy(
            table_hbm.at[idx_v.at[pl.ds(g * ch, ch)]], rows[k], gsems[k])

    def s_copy(g, k):
        return pltpu.make_async_copy(
            rows[k], out_hbm.at[pl.ds(base + g * ch, ch)], osems[k])

    g_copy(0, 0).start()
    g_copy(1, 1).start()

    def body(i, carry):
        a = _NB * i
        for k in range(_NB):
            kp = (k + 2) % _NB
            g_copy(a + k, k).wait()
            s_copy(a + k, k).start()

            @pl.when(a + k + 2 < n_chunks)
            def _():
                @pl.when(a + k >= 2)
                def _():
                    s_copy(a + k - 2, kp).wait()

                g_copy(a + k + 2, kp).start()

        return carry

    lax.fori_loop(0, n_chunks // _NB, body, 0, unroll=False)
    for k in range(_NB):
        s_copy(0, k).wait()


def kernel(token_ids, table):
    b, s = token_ids.shape
    _, d = table.shape
    n = b * s
    assert n % (_NW * _NB * _CH) == 0
    per_w = n // _NW

    flat_ids = token_ids.reshape(n).astype(jnp.int32)
    mesh = plsc.VectorSubcoreMesh(core_axis_name="c", subcore_axis_name="s")
    k = pl.kernel(
        functools.partial(_gather_body, per_w),
        out_type=jax.ShapeDtypeStruct((n, d), jnp.float32),
        mesh=mesh,
        scratch_types=[
            pltpu.VMEM((per_w,), jnp.int32),
            [pltpu.VMEM((_CH, d), jnp.float32) for _ in range(_NB)],
            [pltpu.SemaphoreType.DMA for _ in range(_NB)],
            [pltpu.SemaphoreType.DMA for _ in range(_NB)],
        ],
        compiler_params=pltpu.CompilerParams(use_tc_tiling_on_sc=False),
    )
    out = k(flat_ids, table)
    return out.reshape(b, s, d)
